# baseline (speedup 1.0000x reference)
import functools

import jax
import jax.numpy as jnp
from jax import lax
from jax.experimental import pallas as pl
from jax.experimental.pallas import tpu as pltpu
from jax.experimental.pallas import tpu_sc as plsc

_KS0 = 0
_KS1 = 42
_KS2 = (0x1BD11BDA ^ 42) & 0xFFFFFFFF

_ALPHA = 0.5
_BIG = 2**30


def _i32(x):
    return jnp.int32(x & 0xFFFFFFFF if isinstance(x, int) and x >= 2**31 else x)


def _round(x0, x1, r):
    x0 = x0 + x1
    x1 = x0 ^ ((x1 << r) | lax.shift_right_logical(x1, 32 - r))
    return x0, x1


def _threefry_xored(hi, lo_plus42):
    x0 = hi
    x1 = lo_plus42
    for r in (13, 15, 26, 6):
        x0, x1 = _round(x0, x1, r)
    x0 = x0 + _i32(_KS1)
    x1 = x1 + _i32(_KS2 + 1)
    for r in (17, 29, 16, 24):
        x0, x1 = _round(x0, x1, r)
    x0 = x0 + _i32(_KS2)
    x1 = x1 + _i32(_KS0 + 2)
    for r in (13, 15, 26, 6):
        x0, x1 = _round(x0, x1, r)
    x1 = x1 + _i32(_KS1 + 3)
    for r in (17, 29, 16, 24):
        x0, x1 = _round(x0, x1, r)
    x0 = x0 + _i32(_KS1)
    x1 = x1 + _i32(_KS2 + 4)
    for r in (13, 15, 26, 6):
        x0, x1 = _round(x0, x1, r)
    x0 = x0 + _i32(_KS2)
    x1 = x1 + _i32(_KS0 + 5)
    return x0 ^ x1


def _sample_cell(b0_ref, w_ref, out_ref, r_ref, bv_ref, bc_ref, *, N, L, SC_CELL, UC):
    b = b0_ref[0] + pl.program_id(0)
    cell = pl.program_id(1)

    half_n = jnp.float32((1.0 - _ALPHA) / N)
    w = w_ref[0]
    r_ref[...] = 1.0 / (_ALPHA * w + half_n + 1e-30)

    n_chunk = N // 128
    s0 = cell * SC_CELL
    p0 = b * N + s0
    hi = lax.shift_right_logical(p0, 32 - L)

    lane = lax.broadcasted_iota(jnp.int32, (8, 128), 1)
    sub = lax.broadcasted_iota(jnp.int32, (8, 128), 0)

    def g_body(g, _):
        x1_base = ((p0 + g * 8 + sub) << L) + (lane + 42)
        neg_inf = jnp.full((8, 128), -jnp.inf, jnp.float32)
        zero = jnp.zeros((8, 128), jnp.int32)

        def hash_chunks(t):
            us = []
            for j in range(UC):
                cbase = (t * UC + j) * 128
                bits = _threefry_xored(hi, x1_base + cbase)
                fb = lax.shift_right_logical(bits, 9) | jnp.int32(0x3F800000)
                us.append(lax.bitcast_convert_type(fb, jnp.float32) - 1.0)
            return tuple(us)

        def track(us, t, bv, bc):
            for j in range(UC):
                cc = t * UC + j
                rj = r_ref[pl.ds(cc, 1), :]
                val = jnp.log(us[j]) * rj
                pred = val > bv
                bv = jnp.maximum(val, bv)
                bc = jnp.where(pred, cc, bc)
            return bv, bc

        def c_body(t, carry):
            bv, bc, us = carry
            new_us = hash_chunks(t)
            bv, bc = track(us, t - 1, bv, bc)
            return bv, bc, new_us

        T = n_chunk // UC
        bv, bc, us = lax.fori_loop(1, T, c_body, (neg_inf, zero, hash_chunks(0)))
        bv, bc = track(us, T - 1, bv, bc)
        bv_ref[pl.ds(g * 8, 8), :] = bv
        bc_ref[pl.ds(g * 8, 8), :] = bc
        return _

    lax.fori_loop(0, SC_CELL // 8, g_body, 0)

    lane_t = lax.broadcasted_iota(jnp.int32, (128, 128), 1)
    for gr in range(SC_CELL // 128):
        bva = bv_ref[pl.ds(gr * 128, 128), :]
        bca = bc_ref[pl.ds(gr * 128, 128), :] * 128 + lane_t
        mx = jnp.max(bva, axis=1, keepdims=True)
        idx = jnp.min(jnp.where(bva == mx, bca, _BIG), axis=1, keepdims=True)
        out_ref[0, 0, :, gr] = idx[:, 0]


_LOG2_COEFS = (1.442695, -0.7213474, 0.48089838, -0.36069208, 0.28855568,
               -0.23972578, 0.20490034, -0.19042076, 0.18392456, -0.10877635)
_SQRT2 = 1.4142135623730951


def _sc_sample(weights, b0, n_samp):
    B, N = weights.shape
    L = N.bit_length() - 1
    UCS = 8
    info = plsc.get_sparse_core_info()
    NW = info.num_cores * info.num_subcores
    NC = info.num_cores
    bpw = B // NW if B % NW == 0 else 0
    half_n = float((1.0 - _ALPHA) / N)
    b0arr = jnp.full((16,), b0, jnp.int32)
    mesh = plsc.VectorSubcoreMesh(core_axis_name="c", subcore_axis_name="s")

    @functools.partial(
        pl.kernel,
        mesh=mesh,
        compiler_params=pltpu.CompilerParams(use_tc_tiling_on_sc=False),
        out_type=jax.ShapeDtypeStruct((B, n_samp), jnp.int32),
        scratch_types=[
            pltpu.VMEM((N,), jnp.float32),
            pltpu.VMEM((n_samp,), jnp.int32),
            pltpu.VMEM((16,), jnp.int32),
        ],
    )
    def k(w_hbm, b0_hbm, out_idx, r_v, idx_v, b0_v):
        wid = lax.axis_index("s") * NC + lax.axis_index("c")
        pltpu.sync_copy(b0_hbm, b0_v)
        lane16 = lax.broadcasted_iota(jnp.int32, (16,), 0)
        b0s = b0_v[...][0]
        for i in range(bpw):
            b = wid * bpw + i
            pltpu.sync_copy(w_hbm.at[b], r_v)

            def r_body(t, _):
                wv = r_v[pl.ds(t * 16, 16)]
                r_v[pl.ds(t * 16, 16)] = 1.0 / (_ALPHA * wv + half_n + 1e-30)
                return _

            lax.fori_loop(0, N // 16, r_body, 0)
            pg = b0s + b

            def s_body(sl, accvec):
                s = (N - n_samp) + sl
                P = (pg << L) + s
                lo42 = (P << L) + 42
                hi = lax.shift_right_logical(P, 32 - L)
                bv0 = jnp.full((16,), -jnp.inf, jnp.float32)
                bc0 = jnp.zeros((16,), jnp.int32)

                def ch_body(ch, carry):
                    bv, bc = carry
                    for j in range(UCS):
                        ct = ch * UCS + j
                        rj = r_v[pl.ds(ct * 16, 16)]
                        x1 = (lo42 + ct * 16) + lane16
                        bits = _threefry_xored(hi, x1)
                        fb = lax.shift_right_logical(bits, 9) | jnp.int32(0x3F800000)
                        u = lax.bitcast_convert_type(fb, jnp.float32) - 1.0
                        ub = lax.bitcast_convert_type(u, jnp.int32)
                        e = lax.shift_right_logical(ub, 23) - 127
                        m = lax.bitcast_convert_type(
                            (ub & jnp.int32(0x7FFFFF)) | jnp.int32(0x3F800000),
                            jnp.float32)
                        p2 = m > _SQRT2
                        m = jnp.where(p2, m * 0.5, m)
                        ef = lax.convert_element_type(e, jnp.float32) + jnp.where(
                            p2, jnp.float32(1.0), jnp.float32(0.0))
                        z = m - 1.0
                        q = jnp.float32(_LOG2_COEFS[-1])
                        for cf in _LOG2_COEFS[-2::-1]:
                            q = q * z + jnp.float32(cf)
                        val = (ef + z * q) * rj
                        pr = val > bv
                        bv = jnp.maximum(val, bv)
                        bc = jnp.where(pr, ct, bc)
                    return bv, bc

                bv, bc = lax.fori_loop(0, N // 16 // UCS, ch_body, (bv0, bc0))
                bva = lax.bitcast_convert_type(bv, jnp.int32)
                cva = bc * 16 + lane16
                kb = bva[0]
                cb = cva[0]
                for j in range(1, 16):
                    pj = bva[j] < kb
                    kb = jnp.where(pj, bva[j], kb)
                    cb = jnp.where(pj, cva[j], cb)
                accvec = jnp.where(lane16 == (sl & 15), cb, accvec)

                @pl.when((sl & 15) == 15)
                def _store():
                    idx_v[pl.ds(sl - 15, 16)] = accvec

                return accvec

            lax.fori_loop(0, n_samp, s_body, jnp.zeros((16,), jnp.int32))
            pltpu.sync_copy(idx_v, out_idx.at[b])

    return k(weights, b0arr)


def _compute_indices(weights, b0, n_cells_tc=None, *, interpret=False):
    B, N = weights.shape
    L = N.bit_length() - 1
    assert N == 2**L
    SC_CELL = 1024
    UC = 16
    n_cells = N // SC_CELL if n_cells_tc is None else n_cells_tc
    w3 = weights.reshape(B, N // 128, 128)
    b0arr = jnp.full((1,), b0, jnp.int32)
    out3 = pl.pallas_call(
        functools.partial(_sample_cell, N=N, L=L, SC_CELL=SC_CELL, UC=UC),
        out_shape=jax.ShapeDtypeStruct((B, n_cells, 128, SC_CELL // 128), jnp.int32),
        grid=(B, n_cells),
        in_specs=[
            pl.BlockSpec(memory_space=pltpu.SMEM),
            pl.BlockSpec((1, N // 128, 128), lambda b, c: (b, 0, 0)),
        ],
        out_specs=pl.BlockSpec((1, 1, 128, SC_CELL // 128), lambda b, c: (b, c, 0, 0)),
        scratch_shapes=[
            pltpu.VMEM((N // 128, 128), jnp.float32),
            pltpu.VMEM((SC_CELL, 128), jnp.float32),
            pltpu.VMEM((SC_CELL, 128), jnp.int32),
        ],
        interpret=interpret,
    )(b0arr, w3)
    return jnp.transpose(out3, (0, 1, 3, 2)).reshape(B, n_cells * SC_CELL)


def _sc_gather(particles, weights, indices):
    B, N, D = particles.shape
    info = plsc.get_sparse_core_info()
    NC, NS = info.num_cores, info.num_subcores
    NW = NC * NS
    CH = 2048
    bpw = B // NW if B % NW == 0 else 0
    half_n = float((1.0 - _ALPHA) / N)
    mesh = plsc.VectorSubcoreMesh(core_axis_name="c", subcore_axis_name="s")

    @functools.partial(
        pl.kernel,
        mesh=mesh,
        compiler_params=pltpu.CompilerParams(use_tc_tiling_on_sc=False),
        out_type=(
            jax.ShapeDtypeStruct((B, N, D), jnp.float32),
            jax.ShapeDtypeStruct((B, N), jnp.float32),
        ),
        scratch_types=[
            pltpu.VMEM((N,), jnp.float32),
            pltpu.VMEM((CH,), jnp.int32),
            pltpu.VMEM((CH,), jnp.float32),
            pltpu.VMEM((CH, D), jnp.float32),
            pltpu.SemaphoreType.DMA,
        ],
    )
    def k(p_hbm, w_hbm, idx_hbm, out_p, out_w, q_v, idx_v, ws_v, rows_v, sem):
        wid = lax.axis_index("s") * NC + lax.axis_index("c")
        for i in range(bpw):
            b = wid * bpw + i

            def chunk_body(ch, acc):
                o = ch * CH
                pltpu.sync_copy(idx_hbm.at[b, pl.ds(o, CH)], idx_v)
                pltpu.async_copy(p_hbm.at[b].at[idx_v], rows_v, sem).wait()
                pltpu.sync_copy(rows_v, out_p.at[b, pl.ds(o, CH)])
                pltpu.async_copy(w_hbm.at[b].at[idx_v], ws_v, sem).wait()

                def w_body(t, acc):
                    ws = ws_v[pl.ds(t * 16, 16)]
                    q = ws / ((_ALPHA * ws + half_n) + 1e-10)
                    q_v[pl.ds(o + t * 16, 16)] = q
                    return acc + q

                return lax.fori_loop(0, CH // 16, w_body, acc)

            lax.fori_loop(0, N // CH, chunk_body, jnp.zeros((16,), jnp.float32))
            pltpu.sync_copy(q_v, out_w.at[b])

    return k(particles, weights, indices)


def _norm_rows_kernel(q_ref, out_ref):
    q = q_ref[...]
    out_ref[...] = q / jnp.sum(q, axis=1, keepdims=True)


def _normalize_rows(q):
    B, N = q.shape
    return pl.pallas_call(
        _norm_rows_kernel,
        out_shape=jax.ShapeDtypeStruct((B, N), jnp.float32),
        grid=(B // 8,),
        in_specs=[pl.BlockSpec((8, N), lambda b: (b, 0))],
        out_specs=pl.BlockSpec((8, N), lambda b: (b, 0)),
    )(q)


_SC_CELLS = 7


def _resample_local(particles, weights, b0=0):
    B, N, D = particles.shape
    n_cells = N // 1024
    k_sc = _SC_CELLS if (B % 32 == 0 and N % 16384 == 0) else 0
    if k_sc:
        idx_sc = _sc_sample(weights, b0, k_sc * 1024)
        idx_tc = _compute_indices(weights, b0, n_cells - k_sc)
        indices = jnp.concatenate([idx_tc, idx_sc], axis=1)
    else:
        indices = _compute_indices(weights, b0)
    new_particles, q = _sc_gather(particles, weights, indices)
    return (new_particles, _normalize_rows(q))


def kernel(particles, weights):
    B, N, D = particles.shape
    devs = jax.devices()
    nd = len(devs)
    while nd > 1 and (B % nd != 0 or (B // nd) % 32 != 0):
        nd -= 1
    if nd > 1:
        P = jax.sharding.PartitionSpec
        mesh = jax.sharding.Mesh(devs[:nd], ("x",))

        def _shard_fn(p, w):
            b0 = lax.axis_index("x") * (B // nd)
            return _resample_local(p, w, b0)

        f = jax.shard_map(
            _shard_fn,
            mesh=mesh,
            in_specs=(P("x", None, None), P("x", None)),
            out_specs=(P("x", None, None), P("x", None)),
            check_vma=False,
        )
        return f(particles, weights)
    return _resample_local(particles, weights)
